# baseline (speedup 1.0000x reference)
import functools

import jax
import jax.numpy as jnp
from jax import lax
from jax.experimental import pallas as pl
from jax.experimental.pallas import tpu as pltpu
from jax.experimental.pallas import tpu_sc as plsc

EMB = 16
NW = 32
LANES = 16


def _sc_body(tgt_hbm, ctx_hbm, table_hbm, dots_hbm, part_hbm,
             tidx_v, cidx_v, trows_v, crows_v, dots_v, part_v, sem):
    per_w = tidx_v.shape[0]
    wid = lax.axis_index("s") * 2 + lax.axis_index("c")
    base = wid * per_w

    pltpu.sync_copy(tgt_hbm.at[pl.ds(base, per_w)], tidx_v)
    pltpu.sync_copy(ctx_hbm.at[pl.ds(base, per_w)], cidx_v)
    ct = pltpu.async_copy(table_hbm.at[tidx_v], trows_v, sem)
    cc = pltpu.async_copy(table_hbm.at[cidx_v], crows_v, sem)
    ct.wait()
    cc.wait()

    def blk(i, ssq):
        rb = i * LANES
        acc = jnp.zeros((LANES,), jnp.float32)
        for r in range(LANES):
            t = trows_v[rb + r, :]
            c = crows_v[rb + r, :]
            p = t * c
            s = jnp.sum(p)
            onehot = (lax.iota(jnp.int32, LANES) == r).astype(jnp.float32)
            acc = acc + s * onehot
        dots_v[pl.ds(rb, LANES)] = acc
        return ssq + acc * acc

    ssq = lax.fori_loop(0, per_w // LANES, blk, jnp.zeros((LANES,), jnp.float32))
    part_v[...] = ssq
    pltpu.sync_copy(dots_v, dots_hbm.at[pl.ds(base, per_w)])
    pltpu.sync_copy(part_v, part_hbm.at[wid])


@functools.partial(jax.jit, static_argnums=(3,))
def _sc_call(tgt, ctx, table, n):
    per_w = n // NW
    mesh = plsc.VectorSubcoreMesh(core_axis_name="c", subcore_axis_name="s")
    kern = functools.partial(
        pl.kernel,
        out_type=[
            jax.ShapeDtypeStruct((n,), jnp.float32),
            jax.ShapeDtypeStruct((NW, LANES), jnp.float32),
        ],
        mesh=mesh,
        compiler_params=pltpu.CompilerParams(
            needs_layout_passes=False, use_tc_tiling_on_sc=False),
        scratch_types=[
            pltpu.VMEM((per_w,), jnp.int32),
            pltpu.VMEM((per_w,), jnp.int32),
            pltpu.VMEM((per_w, EMB), jnp.float32),
            pltpu.VMEM((per_w, EMB), jnp.float32),
            pltpu.VMEM((per_w,), jnp.float32),
            pltpu.VMEM((LANES,), jnp.float32),
            pltpu.SemaphoreType.DMA,
        ],
    )(_sc_body)
    return kern(tgt, ctx, table)


def _tc_body(dots_ref, part_ref, out_ref):
    ssq = jnp.sum(part_ref[...])
    scale = lax.rsqrt(jnp.maximum(ssq, 1e-12))
    out_ref[...] = dots_ref[...] * scale


def kernel(target, context, emb_table):
    b, l = target.shape
    n = b * l
    tgt = target.reshape(n).astype(jnp.int32)
    ctx = context.reshape(n).astype(jnp.int32)
    dots, part = _sc_call(tgt, ctx, emb_table, n)
    cos = pl.pallas_call(
        _tc_body,
        out_shape=jax.ShapeDtypeStruct((n // 128, 128), jnp.float32),
    )(dots.reshape(n // 128, 128), part)
    return cos.reshape(b, l)


# speedup vs baseline: 2.3219x; 2.3219x over previous
import functools

import jax
import jax.numpy as jnp
from jax import lax
from jax.experimental import pallas as pl
from jax.experimental.pallas import tpu as pltpu
from jax.experimental.pallas import tpu_sc as plsc

EMB = 16
NW = 32
LANES = 16
CH = 128
UV = 512
NFULL = 1953
VTAIL = 999936


def _sc_params():
    return pltpu.CompilerParams(
        needs_layout_passes=False, use_tc_tiling_on_sc=True)


def _perms():
    lanes = lax.iota(jnp.int32, LANES)
    perm = [(lanes + k) & 15 for k in range(LANES)]
    permhi = [p >> 3 for p in perm]
    return lanes, perm, permhi


def _tr_body(tblt_hbm, tblq_hbm, slab_v, unit_v, tail_slab_v, tail_unit_v,
             sem_in, sem_out):
    wid = lax.axis_index("s") * 2 + lax.axis_index("c")
    n_units = (NFULL - wid + 31) // 32
    lanes, perm, permhi = _perms()

    def load(k):
        u = wid + k * 32
        pltpu.async_copy(
            tblt_hbm.at[:, pl.ds(u * UV, UV)], slab_v.at[lax.rem(k, 2)],
            sem_in)

    def load_wait(k):
        u = wid + k * 32
        pltpu.make_async_copy(
            tblt_hbm.at[:, pl.ds(u * UV, UV)], slab_v.at[lax.rem(k, 2)],
            sem_in).wait()

    def store(k):
        u = wid + k * 32
        pltpu.async_copy(
            unit_v.at[lax.rem(k, 2)], tblq_hbm.at[pl.ds(u * 64, 64)],
            sem_out)

    def store_wait(k):
        u = wid + k * 32
        pltpu.make_async_copy(
            unit_v.at[lax.rem(k, 2)], tblq_hbm.at[pl.ds(u * 64, 64)],
            sem_out).wait()

    @pl.when(n_units > 0)
    def _():
        load(0)

    def unit_step(k, carry):
        @pl.when(k + 1 < n_units)
        def _():
            load(k + 1)

        @pl.when(k >= 2)
        def _():
            store_wait(k - 2)

        load_wait(k)
        par = lax.rem(k, 2)
        pvec = jnp.full((LANES,), par, jnp.int32)

        def vblock(vb, c2):
            vb16 = vb * LANES
            for j in range(LANES):
                src_cols = vb16 + perm[j]
                val = plsc.load_gather(slab_v, [pvec, lanes, src_cols])
                rows = vb * 2 + permhi[j]
                cols = ((perm[j] & 7) << 4) + lanes
                plsc.store_scatter(unit_v, [pvec, rows, cols], val)
            return c2

        lax.fori_loop(0, UV // LANES, vblock, 0)
        store(k)
        return carry

    lax.fori_loop(0, n_units, unit_step, 0)

    @pl.when(n_units >= 2)
    def _():
        store_wait(n_units - 2)

    @pl.when(n_units >= 1)
    def _():
        store_wait(n_units - 1)

    @pl.when(wid == 0)
    def _():
        pltpu.sync_copy(tblt_hbm.at[:, pl.ds(VTAIL, 64)], tail_slab_v)
        for vb in range(4):
            for j in range(LANES):
                src_cols = vb * LANES + perm[j]
                val = plsc.load_gather(tail_slab_v, [lanes, src_cols])
                rows = vb * 2 + permhi[j]
                cols = ((perm[j] & 7) << 4) + lanes
                plsc.store_scatter(tail_unit_v, [rows, cols], val)
        pltpu.sync_copy(tail_unit_v, tblq_hbm.at[pl.ds(VTAIL // 8, 8)])


@jax.jit
def _tr_call(tbl_t):
    mesh = plsc.VectorSubcoreMesh(core_axis_name="c", subcore_axis_name="s")
    kern = functools.partial(
        pl.kernel,
        out_type=jax.ShapeDtypeStruct((125000, 128), jnp.float32),
        mesh=mesh,
        compiler_params=_sc_params(),
        scratch_types=[
            pltpu.VMEM((2, EMB, UV), jnp.float32),
            pltpu.VMEM((2, 64, 128), jnp.float32),
            pltpu.VMEM((EMB, 64), jnp.float32),
            pltpu.VMEM((8, 128), jnp.float32),
            pltpu.SemaphoreType.DMA,
            pltpu.SemaphoreType.DMA,
        ],
    )(_tr_body)
    return kern(tbl_t)


def _sc_body(tgt_hbm, ctx_hbm, tblq_hbm, dots_hbm, part_hbm,
             tidx_v, cidx_v, trow_v, crow_v, tbuf_v, cbuf_v,
             dots_v, part_v, sem):
    per_w = tidx_v.shape[0]
    n_chunks = per_w // CH
    wid = lax.axis_index("s") * 2 + lax.axis_index("c")
    base = wid * per_w

    pltpu.sync_copy(tgt_hbm.at[pl.ds(base, per_w)], tidx_v)
    pltpu.sync_copy(ctx_hbm.at[pl.ds(base, per_w)], cidx_v)

    def shift(i, carry):
        off = i * LANES
        trow_v[pl.ds(off, LANES)] = lax.shift_right_logical(
            tidx_v[pl.ds(off, LANES)], 3)
        crow_v[pl.ds(off, LANES)] = lax.shift_right_logical(
            cidx_v[pl.ds(off, LANES)], 3)
        return carry

    lax.fori_loop(0, per_w // LANES, shift, 0)

    def fire(c):
        par = lax.rem(c, 2)
        pltpu.async_copy(
            tblq_hbm.at[trow_v.at[pl.ds(c * CH, CH)]], tbuf_v.at[par], sem)
        pltpu.async_copy(
            tblq_hbm.at[crow_v.at[pl.ds(c * CH, CH)]], cbuf_v.at[par], sem)

    def wait(c):
        par = lax.rem(c, 2)
        pltpu.make_async_copy(
            tblq_hbm.at[trow_v.at[pl.ds(c * CH, CH)]], tbuf_v.at[par], sem
        ).wait()
        pltpu.make_async_copy(
            tblq_hbm.at[crow_v.at[pl.ds(c * CH, CH)]], cbuf_v.at[par], sem
        ).wait()

    fire(0)
    lanes, perm, _ = _perms()

    def chunk_step(c, ssq):
        @pl.when(c + 1 < n_chunks)
        def _():
            fire(c + 1)
        wait(c)
        par = lax.rem(c, 2)
        pvec = jnp.full((LANES,), par, jnp.int32)
        for b in range(CH // LANES):
            off = c * CH + b * LANES
            tv = tidx_v[pl.ds(off, LANES)]
            cv = cidx_v[pl.ds(off, LANES)]
            tcol = (tv & 7) * EMB
            ccol = (cv & 7) * EMB
            rows = b * LANES + lanes
            acc = jnp.zeros((LANES,), jnp.float32)
            for j in range(EMB):
                t = plsc.load_gather(tbuf_v, [pvec, rows, tcol + perm[j]])
                cc = plsc.load_gather(cbuf_v, [pvec, rows, ccol + perm[j]])
                acc = acc + t * cc
            dots_v[pl.ds(off, LANES)] = acc
            ssq = ssq + acc * acc
        return ssq

    ssq = lax.fori_loop(0, n_chunks, chunk_step,
                        jnp.zeros((LANES,), jnp.float32))
    part_v[...] = ssq
    pltpu.sync_copy(dots_v, dots_hbm.at[pl.ds(base, per_w)])
    pltpu.sync_copy(part_v, part_hbm.at[pl.ds(wid * LANES, LANES)])


@functools.partial(jax.jit, static_argnums=(3,))
def _sc_call(tgt, ctx, tblq, n):
    per_w = n // NW
    mesh = plsc.VectorSubcoreMesh(core_axis_name="c", subcore_axis_name="s")
    kern = functools.partial(
        pl.kernel,
        out_type=[
            jax.ShapeDtypeStruct((n,), jnp.float32),
            jax.ShapeDtypeStruct((NW * LANES,), jnp.float32),
        ],
        mesh=mesh,
        compiler_params=_sc_params(),
        scratch_types=[
            pltpu.VMEM((per_w,), jnp.int32),
            pltpu.VMEM((per_w,), jnp.int32),
            pltpu.VMEM((per_w,), jnp.int32),
            pltpu.VMEM((per_w,), jnp.int32),
            pltpu.VMEM((2, CH, 128), jnp.float32),
            pltpu.VMEM((2, CH, 128), jnp.float32),
            pltpu.VMEM((per_w,), jnp.float32),
            pltpu.VMEM((LANES,), jnp.float32),
            pltpu.SemaphoreType.DMA,
        ],
    )(_sc_body)
    return kern(tgt, ctx, tblq)


def _tc_body(dots_ref, part_ref, out_ref):
    ssq = jnp.sum(part_ref[...])
    scale = lax.rsqrt(jnp.maximum(ssq, 1e-12))
    out_ref[...] = dots_ref[...] * scale


def kernel(target, context, emb_table):
    b, l = target.shape
    n = b * l
    tgt = target.reshape(n).astype(jnp.int32)
    ctx = context.reshape(n).astype(jnp.int32)
    tblq = _tr_call(emb_table.T)
    dots, part = _sc_call(tgt, ctx, tblq, n)
    cos = pl.pallas_call(
        _tc_body,
        out_shape=jax.ShapeDtypeStruct((n // 128, 128), jnp.float32),
    )(dots.reshape(n // 128, 128), part.reshape(4, 128))
    return cos.reshape(b, l)


# speedup vs baseline: 2.5672x; 1.1056x over previous
import functools

import jax
import jax.numpy as jnp
from jax import lax
from jax.experimental import pallas as pl
from jax.experimental.pallas import tpu as pltpu
from jax.experimental.pallas import tpu_sc as plsc

EMB = 16
NW = 32
LANES = 16
CH = 128
UV = 512
NFULL = 1953
VTAIL = 999936


def _sc_params():
    return pltpu.CompilerParams(
        needs_layout_passes=False, use_tc_tiling_on_sc=True)


def _perms():
    lanes = lax.iota(jnp.int32, LANES)
    perm = [(lanes + k) & 15 for k in range(LANES)]
    permhi = [p >> 3 for p in perm]
    return lanes, perm, permhi


def _tr_body(tblt_hbm, tblq_hbm, slab_v, unit_v, tail_slab_v, tail_unit_v,
             sem_in, sem_out):
    wid = lax.axis_index("s") * 2 + lax.axis_index("c")
    n_units = (NFULL - wid + 31) // 32
    lanes, perm, permhi = _perms()

    def load(k):
        u = wid + k * 32
        pltpu.async_copy(
            tblt_hbm.at[:, pl.ds(u * UV, UV)], slab_v.at[lax.rem(k, 2)],
            sem_in)

    def load_wait(k):
        u = wid + k * 32
        pltpu.make_async_copy(
            tblt_hbm.at[:, pl.ds(u * UV, UV)], slab_v.at[lax.rem(k, 2)],
            sem_in).wait()

    def store(k):
        u = wid + k * 32
        pltpu.async_copy(
            unit_v.at[lax.rem(k, 2)], tblq_hbm.at[pl.ds(u * 64, 64)],
            sem_out)

    def store_wait(k):
        u = wid + k * 32
        pltpu.make_async_copy(
            unit_v.at[lax.rem(k, 2)], tblq_hbm.at[pl.ds(u * 64, 64)],
            sem_out).wait()

    @pl.when(n_units > 0)
    def _():
        load(0)

    def unit_step(k, carry):
        @pl.when(k + 1 < n_units)
        def _():
            load(k + 1)

        @pl.when(k >= 2)
        def _():
            store_wait(k - 2)

        load_wait(k)
        par = lax.rem(k, 2)
        pvec = jnp.full((LANES,), par, jnp.int32)

        def vgroup(g, c2):
            for bi in range(4):
                vb = g * 4 + bi
                vb16 = vb * LANES
                for j in range(LANES):
                    src_cols = vb16 + perm[j]
                    val = plsc.load_gather(slab_v, [pvec, lanes, src_cols])
                    rows = vb * 2 + permhi[j]
                    cols = ((perm[j] & 7) << 4) + lanes
                    plsc.store_scatter(unit_v, [pvec, rows, cols], val)
            return c2

        lax.fori_loop(0, UV // LANES // 4, vgroup, 0)
        store(k)
        return carry

    lax.fori_loop(0, n_units, unit_step, 0)

    @pl.when(n_units >= 2)
    def _():
        store_wait(n_units - 2)

    @pl.when(n_units >= 1)
    def _():
        store_wait(n_units - 1)

    @pl.when(wid == 0)
    def _():
        pltpu.sync_copy(tblt_hbm.at[:, pl.ds(VTAIL, 64)], tail_slab_v)
        for vb in range(4):
            for j in range(LANES):
                src_cols = vb * LANES + perm[j]
                val = plsc.load_gather(tail_slab_v, [lanes, src_cols])
                rows = vb * 2 + permhi[j]
                cols = ((perm[j] & 7) << 4) + lanes
                plsc.store_scatter(tail_unit_v, [rows, cols], val)
        pltpu.sync_copy(tail_unit_v, tblq_hbm.at[pl.ds(VTAIL // 8, 8)])


@jax.jit
def _tr_call(tbl_t):
    mesh = plsc.VectorSubcoreMesh(core_axis_name="c", subcore_axis_name="s")
    kern = functools.partial(
        pl.kernel,
        out_type=jax.ShapeDtypeStruct((125000, 128), jnp.float32),
        mesh=mesh,
        compiler_params=_sc_params(),
        scratch_types=[
            pltpu.VMEM((2, EMB, UV), jnp.float32),
            pltpu.VMEM((2, 64, 128), jnp.float32),
            pltpu.VMEM((EMB, 64), jnp.float32),
            pltpu.VMEM((8, 128), jnp.float32),
            pltpu.SemaphoreType.DMA,
            pltpu.SemaphoreType.DMA,
        ],
    )(_tr_body)
    return kern(tbl_t)


def _sc_body(tgt_hbm, ctx_hbm, tblq_hbm, dots_hbm, part_hbm,
             tidx_v, cidx_v, trow_v, crow_v, tbuf_v, cbuf_v,
             dots_v, part_v, sem):
    per_w = tidx_v.shape[0]
    n_chunks = per_w // CH
    wid = lax.axis_index("s") * 2 + lax.axis_index("c")
    base = wid * per_w

    pltpu.sync_copy(tgt_hbm.at[pl.ds(base, per_w)], tidx_v)
    pltpu.sync_copy(ctx_hbm.at[pl.ds(base, per_w)], cidx_v)

    def shift(i, carry):
        off = i * LANES
        trow_v[pl.ds(off, LANES)] = lax.shift_right_logical(
            tidx_v[pl.ds(off, LANES)], 3)
        crow_v[pl.ds(off, LANES)] = lax.shift_right_logical(
            cidx_v[pl.ds(off, LANES)], 3)
        return carry

    lax.fori_loop(0, per_w // LANES, shift, 0)

    def fire(c):
        par = lax.rem(c, 2)
        pltpu.async_copy(
            tblq_hbm.at[trow_v.at[pl.ds(c * CH, CH)]], tbuf_v.at[par], sem)
        pltpu.async_copy(
            tblq_hbm.at[crow_v.at[pl.ds(c * CH, CH)]], cbuf_v.at[par], sem)

    def wait(c):
        par = lax.rem(c, 2)
        pltpu.make_async_copy(
            tblq_hbm.at[trow_v.at[pl.ds(c * CH, CH)]], tbuf_v.at[par], sem
        ).wait()
        pltpu.make_async_copy(
            tblq_hbm.at[crow_v.at[pl.ds(c * CH, CH)]], cbuf_v.at[par], sem
        ).wait()

    fire(0)
    lanes, perm, _ = _perms()

    def chunk_step(c, ssq):
        @pl.when(c + 1 < n_chunks)
        def _():
            fire(c + 1)
        wait(c)
        par = lax.rem(c, 2)
        pvec = jnp.full((LANES,), par, jnp.int32)
        for b in range(CH // LANES):
            off = c * CH + b * LANES
            tv = tidx_v[pl.ds(off, LANES)]
            cv = cidx_v[pl.ds(off, LANES)]
            tcol = (tv & 7) * EMB
            ccol = (cv & 7) * EMB
            rows = b * LANES + lanes
            acc = jnp.zeros((LANES,), jnp.float32)
            for j in range(EMB):
                t = plsc.load_gather(tbuf_v, [pvec, rows, tcol + perm[j]])
                cc = plsc.load_gather(cbuf_v, [pvec, rows, ccol + perm[j]])
                acc = acc + t * cc
            dots_v[pl.ds(off, LANES)] = acc
            ssq = ssq + acc * acc
        return ssq

    ssq = lax.fori_loop(0, n_chunks, chunk_step,
                        jnp.zeros((LANES,), jnp.float32))
    part_v[...] = ssq
    pltpu.sync_copy(dots_v, dots_hbm.at[pl.ds(base, per_w)])
    pltpu.sync_copy(part_v, part_hbm.at[pl.ds(wid * LANES, LANES)])


@functools.partial(jax.jit, static_argnums=(3,))
def _sc_call(tgt, ctx, tblq, n):
    per_w = n // NW
    mesh = plsc.VectorSubcoreMesh(core_axis_name="c", subcore_axis_name="s")
    kern = functools.partial(
        pl.kernel,
        out_type=[
            jax.ShapeDtypeStruct((n,), jnp.float32),
            jax.ShapeDtypeStruct((NW * LANES,), jnp.float32),
        ],
        mesh=mesh,
        compiler_params=_sc_params(),
        scratch_types=[
            pltpu.VMEM((per_w,), jnp.int32),
            pltpu.VMEM((per_w,), jnp.int32),
            pltpu.VMEM((per_w,), jnp.int32),
            pltpu.VMEM((per_w,), jnp.int32),
            pltpu.VMEM((2, CH, 128), jnp.float32),
            pltpu.VMEM((2, CH, 128), jnp.float32),
            pltpu.VMEM((per_w,), jnp.float32),
            pltpu.VMEM((LANES,), jnp.float32),
            pltpu.SemaphoreType.DMA,
        ],
    )(_sc_body)
    return kern(tgt, ctx, tblq)


def _tc_body(dots_ref, part_ref, out_ref):
    ssq = jnp.sum(part_ref[...])
    scale = lax.rsqrt(jnp.maximum(ssq, 1e-12))
    out_ref[...] = dots_ref[...] * scale


def kernel(target, context, emb_table):
    b, l = target.shape
    n = b * l
    tgt = target.T.reshape(n).astype(jnp.int32)
    ctx = context.T.reshape(n).astype(jnp.int32)
    tblq = _tr_call(emb_table.T)
    dots, part = _sc_call(tgt, ctx, tblq, n)
    cos = pl.pallas_call(
        _tc_body,
        out_shape=jax.ShapeDtypeStruct((n // 128, 128), jnp.float32),
    )(dots.reshape(n // 128, 128), part.reshape(4, 128))
    return cos.reshape(l, b).T


# speedup vs baseline: 3.8275x; 1.4910x over previous
import functools

import jax
import jax.numpy as jnp
from jax import lax
from jax.experimental import pallas as pl
from jax.experimental.pallas import tpu as pltpu
from jax.experimental.pallas import tpu_sc as plsc

EMB = 16
NW = 32
LANES = 16
CH = 128
UV = 512
NFULL = 1953
VTAIL = 999936


def _sc_params():
    return pltpu.CompilerParams(
        needs_layout_passes=False, use_tc_tiling_on_sc=True)


def _perms():
    lanes = lax.iota(jnp.int32, LANES)
    perm = [(lanes + k) & 15 for k in range(LANES)]
    permhi = [p >> 3 for p in perm]
    return lanes, perm, permhi


def _tr_body(tblt_hbm, tblq_hbm, slab_v, unit_v, tail_slab_v, tail_unit_v,
             sem_in, sem_out):
    wid = lax.axis_index("s") * 2 + lax.axis_index("c")
    n_units = (NFULL - wid + 31) // 32
    lanes, perm, permhi = _perms()

    def load(k):
        u = wid + k * 32
        pltpu.async_copy(
            tblt_hbm.at[:, pl.ds(u * UV, UV)], slab_v.at[lax.rem(k, 2)],
            sem_in)

    def load_wait(k):
        u = wid + k * 32
        pltpu.make_async_copy(
            tblt_hbm.at[:, pl.ds(u * UV, UV)], slab_v.at[lax.rem(k, 2)],
            sem_in).wait()

    def store(k):
        u = wid + k * 32
        pltpu.async_copy(
            unit_v.at[lax.rem(k, 2)], tblq_hbm.at[pl.ds(u * 64, 64)],
            sem_out)

    def store_wait(k):
        u = wid + k * 32
        pltpu.make_async_copy(
            unit_v.at[lax.rem(k, 2)], tblq_hbm.at[pl.ds(u * 64, 64)],
            sem_out).wait()

    @pl.when(n_units > 0)
    def _():
        load(0)

    def unit_step(k, carry):
        @pl.when(k + 1 < n_units)
        def _():
            load(k + 1)

        @pl.when(k >= 2)
        def _():
            store_wait(k - 2)

        load_wait(k)
        par = lax.rem(k, 2)
        pvec = jnp.full((LANES,), par, jnp.int32)

        def vgroup(g, c2):
            for bi in range(4):
                vb = g * 4 + bi
                vb16 = vb * LANES
                for jj in range(0, LANES, 4):
                    vals = [
                        plsc.load_gather(slab_v, [pvec, lanes,
                                                  vb16 + perm[jj + t]])
                        for t in range(4)
                    ]
                    for t in range(4):
                        j = jj + t
                        rows = vb * 2 + permhi[j]
                        cols = ((perm[j] & 7) << 4) + lanes
                        plsc.store_scatter(unit_v, [pvec, rows, cols],
                                           vals[t])
            return c2

        lax.fori_loop(0, UV // LANES // 4, vgroup, 0)
        store(k)
        return carry

    lax.fori_loop(0, n_units, unit_step, 0)

    @pl.when(n_units >= 2)
    def _():
        store_wait(n_units - 2)

    @pl.when(n_units >= 1)
    def _():
        store_wait(n_units - 1)

    @pl.when(wid == 0)
    def _():
        pltpu.sync_copy(tblt_hbm.at[:, pl.ds(VTAIL, 64)], tail_slab_v)
        for vb in range(4):
            for j in range(LANES):
                src_cols = vb * LANES + perm[j]
                val = plsc.load_gather(tail_slab_v, [lanes, src_cols])
                rows = vb * 2 + permhi[j]
                cols = ((perm[j] & 7) << 4) + lanes
                plsc.store_scatter(tail_unit_v, [rows, cols], val)
        pltpu.sync_copy(tail_unit_v, tblq_hbm.at[pl.ds(VTAIL // 8, 8)])


@jax.jit
def _tr_call(tbl_t):
    mesh = plsc.VectorSubcoreMesh(core_axis_name="c", subcore_axis_name="s")
    kern = functools.partial(
        pl.kernel,
        out_type=jax.ShapeDtypeStruct((125000, 128), jnp.float32),
        mesh=mesh,
        compiler_params=_sc_params(),
        scratch_types=[
            pltpu.VMEM((2, EMB, UV), jnp.float32),
            pltpu.VMEM((2, 64, 128), jnp.float32),
            pltpu.VMEM((EMB, 64), jnp.float32),
            pltpu.VMEM((8, 128), jnp.float32),
            pltpu.SemaphoreType.DMA,
            pltpu.SemaphoreType.DMA,
        ],
    )(_tr_body)
    return kern(tbl_t)


def _sc_body(tgt_hbm, ctx_hbm, tblq_hbm, dots_hbm, part_hbm,
             tidx_v, cidx_v, trow_v, crow_v, tbuf_v, cbuf_v,
             dots_v, part_v, sem):
    per_w = tidx_v.shape[0]
    n_chunks = per_w // CH
    wid = lax.axis_index("s") * 2 + lax.axis_index("c")
    base = wid * per_w

    pltpu.sync_copy(tgt_hbm.at[pl.ds(base, per_w)], tidx_v)
    pltpu.sync_copy(ctx_hbm.at[pl.ds(base, per_w)], cidx_v)

    def shift(i, carry):
        off = i * LANES
        trow_v[pl.ds(off, LANES)] = lax.shift_right_logical(
            tidx_v[pl.ds(off, LANES)], 3)
        crow_v[pl.ds(off, LANES)] = lax.shift_right_logical(
            cidx_v[pl.ds(off, LANES)], 3)
        return carry

    lax.fori_loop(0, per_w // LANES, shift, 0)

    def fire(c):
        par = lax.rem(c, 2)
        pltpu.async_copy(
            tblq_hbm.at[trow_v.at[pl.ds(c * CH, CH)]], tbuf_v.at[par], sem)
        pltpu.async_copy(
            tblq_hbm.at[crow_v.at[pl.ds(c * CH, CH)]], cbuf_v.at[par], sem)

    def wait(c):
        par = lax.rem(c, 2)
        pltpu.make_async_copy(
            tblq_hbm.at[trow_v.at[pl.ds(c * CH, CH)]], tbuf_v.at[par], sem
        ).wait()
        pltpu.make_async_copy(
            tblq_hbm.at[crow_v.at[pl.ds(c * CH, CH)]], cbuf_v.at[par], sem
        ).wait()

    fire(0)
    lanes, perm, _ = _perms()

    def chunk_step(c, ssq):
        @pl.when(c + 1 < n_chunks)
        def _():
            fire(c + 1)
        wait(c)
        par = lax.rem(c, 2)
        pvec = jnp.full((LANES,), par, jnp.int32)
        for b in range(CH // LANES):
            off = c * CH + b * LANES
            tv = tidx_v[pl.ds(off, LANES)]
            cv = cidx_v[pl.ds(off, LANES)]
            tcol = (tv & 7) * EMB
            ccol = (cv & 7) * EMB
            rows = b * LANES + lanes
            acc = jnp.zeros((LANES,), jnp.float32)
            for j in range(EMB):
                t = plsc.load_gather(tbuf_v, [pvec, rows, tcol + perm[j]])
                cc = plsc.load_gather(cbuf_v, [pvec, rows, ccol + perm[j]])
                acc = acc + t * cc
            dots_v[pl.ds(off, LANES)] = acc
            ssq = ssq + acc * acc
        return ssq

    ssq = lax.fori_loop(0, n_chunks, chunk_step,
                        jnp.zeros((LANES,), jnp.float32))
    part_v[...] = ssq
    pltpu.sync_copy(dots_v, dots_hbm.at[pl.ds(base, per_w)])
    pltpu.sync_copy(part_v, part_hbm.at[pl.ds(wid * LANES, LANES)])


@functools.partial(jax.jit, static_argnums=(3,))
def _sc_call(tgt, ctx, tblq, n):
    per_w = n // NW
    mesh = plsc.VectorSubcoreMesh(core_axis_name="c", subcore_axis_name="s")
    kern = functools.partial(
        pl.kernel,
        out_type=[
            jax.ShapeDtypeStruct((n,), jnp.float32),
            jax.ShapeDtypeStruct((NW * LANES,), jnp.float32),
        ],
        mesh=mesh,
        compiler_params=_sc_params(),
        scratch_types=[
            pltpu.VMEM((per_w,), jnp.int32),
            pltpu.VMEM((per_w,), jnp.int32),
            pltpu.VMEM((per_w,), jnp.int32),
            pltpu.VMEM((per_w,), jnp.int32),
            pltpu.VMEM((2, CH, 128), jnp.float32),
            pltpu.VMEM((2, CH, 128), jnp.float32),
            pltpu.VMEM((per_w,), jnp.float32),
            pltpu.VMEM((LANES,), jnp.float32),
            pltpu.SemaphoreType.DMA,
        ],
    )(_sc_body)
    return kern(tgt, ctx, tblq)


def _tc_body(dots_ref, part_ref, out_ref):
    ssq = jnp.sum(part_ref[...])
    scale = lax.rsqrt(jnp.maximum(ssq, 1e-12))
    out_ref[...] = dots_ref[...] * scale


def kernel(target, context, emb_table):
    b, l = target.shape
    n = b * l
    tgt = target.T.reshape(n).astype(jnp.int32)
    ctx = context.T.reshape(n).astype(jnp.int32)
    tblq = _tr_call(emb_table.T)
    dots, part = _sc_call(tgt, ctx, tblq, n)
    cos = pl.pallas_call(
        _tc_body,
        out_shape=jax.ShapeDtypeStruct((n // 128, 128), jnp.float32),
    )(dots.reshape(n // 128, 128), part.reshape(4, 128))
    return cos.reshape(l, b).T


# speedup vs baseline: 4.4733x; 1.1687x over previous
import functools

import jax
import jax.numpy as jnp
from jax import lax
from jax.experimental import pallas as pl
from jax.experimental.pallas import tpu as pltpu
from jax.experimental.pallas import tpu_sc as plsc

EMB = 16
NW = 32
LANES = 16
CH = 128
UV = 512
NFULL = 1953
VTAIL = 999936


def _sc_params():
    return pltpu.CompilerParams(
        needs_layout_passes=False, use_tc_tiling_on_sc=True)


def _perms():
    lanes = lax.iota(jnp.int32, LANES)
    perm = [(lanes + k) & 15 for k in range(LANES)]
    permhi = [p >> 3 for p in perm]
    return lanes, perm, permhi


def _tr_body(tblt_hbm, tblq_hbm, slab_v, unit_v, tail_slab_v, tail_unit_v,
             sem_in, sem_out):
    wid = lax.axis_index("s") * 2 + lax.axis_index("c")
    n_units = (NFULL - wid + 31) // 32
    lanes, perm, permhi = _perms()

    def load(k):
        u = wid + k * 32
        pltpu.async_copy(
            tblt_hbm.at[:, pl.ds(u * UV, UV)], slab_v.at[lax.rem(k, 2)],
            sem_in)

    def load_wait(k):
        u = wid + k * 32
        pltpu.make_async_copy(
            tblt_hbm.at[:, pl.ds(u * UV, UV)], slab_v.at[lax.rem(k, 2)],
            sem_in).wait()

    def store(k):
        u = wid + k * 32
        pltpu.async_copy(
            unit_v.at[lax.rem(k, 2)], tblq_hbm.at[pl.ds(u * 64, 64)],
            sem_out)

    def store_wait(k):
        u = wid + k * 32
        pltpu.make_async_copy(
            unit_v.at[lax.rem(k, 2)], tblq_hbm.at[pl.ds(u * 64, 64)],
            sem_out).wait()

    @pl.when(n_units > 0)
    def _():
        load(0)

    def unit_step(k, carry):
        @pl.when(k + 1 < n_units)
        def _():
            load(k + 1)

        @pl.when(k >= 2)
        def _():
            store_wait(k - 2)

        load_wait(k)
        par = lax.rem(k, 2)
        pvec = jnp.full((LANES,), par, jnp.int32)

        def vgroup(g, c2):
            for bi in range(4):
                vb = g * 4 + bi
                vb16 = vb * LANES
                for jj in range(0, LANES, 4):
                    vals = [
                        plsc.load_gather(slab_v, [pvec, lanes,
                                                  vb16 + perm[jj + t]])
                        for t in range(4)
                    ]
                    for t in range(4):
                        j = jj + t
                        rows = vb * 2 + permhi[j]
                        cols = ((perm[j] & 7) << 4) + lanes
                        plsc.store_scatter(unit_v, [pvec, rows, cols],
                                           vals[t])
            return c2

        lax.fori_loop(0, UV // LANES // 4, vgroup, 0)
        store(k)
        return carry

    lax.fori_loop(0, n_units, unit_step, 0)

    @pl.when(n_units >= 2)
    def _():
        store_wait(n_units - 2)

    @pl.when(n_units >= 1)
    def _():
        store_wait(n_units - 1)

    @pl.when(wid == 0)
    def _():
        pltpu.sync_copy(tblt_hbm.at[:, pl.ds(VTAIL, 64)], tail_slab_v)
        for vb in range(4):
            for j in range(LANES):
                src_cols = vb * LANES + perm[j]
                val = plsc.load_gather(tail_slab_v, [lanes, src_cols])
                rows = vb * 2 + permhi[j]
                cols = ((perm[j] & 7) << 4) + lanes
                plsc.store_scatter(tail_unit_v, [rows, cols], val)
        pltpu.sync_copy(tail_unit_v, tblq_hbm.at[pl.ds(VTAIL // 8, 8)])


@jax.jit
def _tr_call(tbl_t):
    mesh = plsc.VectorSubcoreMesh(core_axis_name="c", subcore_axis_name="s")
    kern = functools.partial(
        pl.kernel,
        out_type=jax.ShapeDtypeStruct((125000, 128), jnp.float32),
        mesh=mesh,
        compiler_params=_sc_params(),
        scratch_types=[
            pltpu.VMEM((2, EMB, UV), jnp.float32),
            pltpu.VMEM((2, 64, 128), jnp.float32),
            pltpu.VMEM((EMB, 64), jnp.float32),
            pltpu.VMEM((8, 128), jnp.float32),
            pltpu.SemaphoreType.DMA,
            pltpu.SemaphoreType.DMA,
        ],
    )(_tr_body)
    return kern(tbl_t)


def _sc_body(tgt_hbm, ctx_hbm, tbl_hbm, dots_hbm, part_hbm,
             tidx_v, cidx_v, trows_v, crows_v, dots_v, part_v, sem):
    per_w = tidx_v.shape[0]
    wid = lax.axis_index("s") * 2 + lax.axis_index("c")
    base = wid * per_w

    pltpu.sync_copy(tgt_hbm.at[pl.ds(base, per_w)], tidx_v)
    pltpu.sync_copy(ctx_hbm.at[pl.ds(base, per_w)], cidx_v)
    ct = pltpu.async_copy(tbl_hbm.at[tidx_v], trows_v, sem)
    cc = pltpu.async_copy(tbl_hbm.at[cidx_v], crows_v, sem)
    ct.wait()
    cc.wait()

    lanes, perm, _ = _perms()

    def blk(i, ssq):
        rb = i * LANES
        rows = rb + lanes
        acc = jnp.zeros((LANES,), jnp.float32)
        for j in range(EMB):
            t = plsc.load_gather(trows_v, [rows, perm[j]])
            c = plsc.load_gather(crows_v, [rows, perm[j]])
            acc = acc + t * c
        dots_v[pl.ds(rb, LANES)] = acc
        return ssq + acc * acc

    ssq = lax.fori_loop(0, per_w // LANES, blk,
                        jnp.zeros((LANES,), jnp.float32))
    part_v[...] = ssq
    pltpu.sync_copy(dots_v, dots_hbm.at[pl.ds(base, per_w)])
    pltpu.sync_copy(part_v, part_hbm.at[pl.ds(wid * LANES, LANES)])


@functools.partial(jax.jit, static_argnums=(3,))
def _sc_call(tgt, ctx, tbl, n):
    per_w = n // NW
    mesh = plsc.VectorSubcoreMesh(core_axis_name="c", subcore_axis_name="s")
    kern = functools.partial(
        pl.kernel,
        out_type=[
            jax.ShapeDtypeStruct((n,), jnp.float32),
            jax.ShapeDtypeStruct((NW * LANES,), jnp.float32),
        ],
        mesh=mesh,
        compiler_params=pltpu.CompilerParams(
            needs_layout_passes=False, use_tc_tiling_on_sc=False),
        scratch_types=[
            pltpu.VMEM((per_w,), jnp.int32),
            pltpu.VMEM((per_w,), jnp.int32),
            pltpu.VMEM((per_w, EMB), jnp.float32),
            pltpu.VMEM((per_w, EMB), jnp.float32),
            pltpu.VMEM((per_w,), jnp.float32),
            pltpu.VMEM((LANES,), jnp.float32),
            pltpu.SemaphoreType.DMA,
        ],
    )(_sc_body)
    return kern(tgt, ctx, tbl)


def _tc_body(dots_ref, part_ref, out_ref):
    ssq = jnp.sum(part_ref[...])
    scale = lax.rsqrt(jnp.maximum(ssq, 1e-12))
    out_ref[...] = dots_ref[...] * scale


def kernel(target, context, emb_table):
    b, l = target.shape
    n = b * l
    tgt = target.T.reshape(n).astype(jnp.int32)
    ctx = context.T.reshape(n).astype(jnp.int32)
    tblq = _tr_call(emb_table.T)
    dots, part = _sc_call(tgt, ctx, tblq.reshape(emb_table.shape[0], EMB), n)
    cos = pl.pallas_call(
        _tc_body,
        out_shape=jax.ShapeDtypeStruct((n // 128, 128), jnp.float32),
    )(dots.reshape(n // 128, 128), part.reshape(4, 128))
    return cos.reshape(l, b).T


# speedup vs baseline: 4.4846x; 1.0025x over previous
import functools

import jax
import jax.numpy as jnp
from jax import lax
from jax.experimental import pallas as pl
from jax.experimental.pallas import tpu as pltpu
from jax.experimental.pallas import tpu_sc as plsc

EMB = 16
NW = 32
LANES = 16
CH = 128
UV = 512
NFULL = 1953
VTAIL = 999936


def _sc_params():
    return pltpu.CompilerParams(
        needs_layout_passes=False, use_tc_tiling_on_sc=True)


def _perms():
    lanes = lax.iota(jnp.int32, LANES)
    perm = [(lanes + k) & 15 for k in range(LANES)]
    permhi = [p >> 3 for p in perm]
    return lanes, perm, permhi


def _tr_body(tblt_hbm, tblq_hbm, slab_v, unit_v, tail_slab_v, tail_unit_v,
             sem_in, sem_out):
    wid = lax.axis_index("s") * 2 + lax.axis_index("c")
    n_units = (NFULL - wid + 31) // 32
    lanes, perm, permhi = _perms()

    def load(k):
        u = wid + k * 32
        pltpu.async_copy(
            tblt_hbm.at[:, pl.ds(u * UV, UV)], slab_v.at[lax.rem(k, 2)],
            sem_in)

    def load_wait(k):
        u = wid + k * 32
        pltpu.make_async_copy(
            tblt_hbm.at[:, pl.ds(u * UV, UV)], slab_v.at[lax.rem(k, 2)],
            sem_in).wait()

    def store(k):
        u = wid + k * 32
        pltpu.async_copy(
            unit_v.at[lax.rem(k, 4)], tblq_hbm.at[pl.ds(u * 64, 64)],
            sem_out)

    def store_wait(k):
        u = wid + k * 32
        pltpu.make_async_copy(
            unit_v.at[lax.rem(k, 4)], tblq_hbm.at[pl.ds(u * 64, 64)],
            sem_out).wait()

    @pl.when(n_units > 0)
    def _():
        load(0)

    def unit_step(k, carry):
        @pl.when(k + 1 < n_units)
        def _():
            load(k + 1)

        @pl.when(k >= 4)
        def _():
            store_wait(k - 4)

        load_wait(k)
        par = lax.rem(k, 2)
        upar = lax.rem(k, 4)
        pvec = jnp.full((LANES,), par, jnp.int32)
        uvec = jnp.full((LANES,), upar, jnp.int32)

        def vgroup(g, c2):
            for bi in range(4):
                vb = g * 4 + bi
                vb16 = vb * LANES
                for jj in range(0, LANES, 4):
                    vals = [
                        plsc.load_gather(slab_v, [pvec, lanes,
                                                  vb16 + perm[jj + t]])
                        for t in range(4)
                    ]
                    for t in range(4):
                        j = jj + t
                        rows = vb * 2 + permhi[j]
                        cols = ((perm[j] & 7) << 4) + lanes
                        plsc.store_scatter(unit_v, [uvec, rows, cols],
                                           vals[t])
            return c2

        lax.fori_loop(0, UV // LANES // 4, vgroup, 0)
        store(k)
        return carry

    lax.fori_loop(0, n_units, unit_step, 0)

    def drain(k, carry):
        @pl.when(k >= 0)
        def _():
            store_wait(k)
        return carry

    lax.fori_loop(lax.max(n_units - 4, 0), n_units, drain, 0)

    @pl.when(wid == 0)
    def _():
        pltpu.sync_copy(tblt_hbm.at[:, pl.ds(VTAIL, 64)], tail_slab_v)
        for vb in range(4):
            for j in range(LANES):
                src_cols = vb * LANES + perm[j]
                val = plsc.load_gather(tail_slab_v, [lanes, src_cols])
                rows = vb * 2 + permhi[j]
                cols = ((perm[j] & 7) << 4) + lanes
                plsc.store_scatter(tail_unit_v, [rows, cols], val)
        pltpu.sync_copy(tail_unit_v, tblq_hbm.at[pl.ds(VTAIL // 8, 8)])


@jax.jit
def _tr_call(tbl_t):
    mesh = plsc.VectorSubcoreMesh(core_axis_name="c", subcore_axis_name="s")
    kern = functools.partial(
        pl.kernel,
        out_type=jax.ShapeDtypeStruct((125000, 128), jnp.float32),
        mesh=mesh,
        compiler_params=_sc_params(),
        scratch_types=[
            pltpu.VMEM((2, EMB, UV), jnp.float32),
            pltpu.VMEM((4, 64, 128), jnp.float32),
            pltpu.VMEM((EMB, 64), jnp.float32),
            pltpu.VMEM((8, 128), jnp.float32),
            pltpu.SemaphoreType.DMA,
            pltpu.SemaphoreType.DMA,
        ],
    )(_tr_body)
    return kern(tbl_t)


def _sc_body(tgt_hbm, ctx_hbm, tbl_hbm, dots_hbm, part_hbm,
             tidx_v, cidx_v, trows_v, crows_v, dots_v, part_v, sem):
    per_w = tidx_v.shape[0]
    wid = lax.axis_index("s") * 2 + lax.axis_index("c")
    base = wid * per_w

    pltpu.sync_copy(tgt_hbm.at[pl.ds(base, per_w)], tidx_v)
    pltpu.sync_copy(ctx_hbm.at[pl.ds(base, per_w)], cidx_v)
    ct = pltpu.async_copy(tbl_hbm.at[tidx_v], trows_v, sem)
    cc = pltpu.async_copy(tbl_hbm.at[cidx_v], crows_v, sem)
    ct.wait()
    cc.wait()

    lanes, perm, _ = _perms()

    def blk(i, ssq):
        rb = i * LANES
        rows = rb + lanes
        acc = jnp.zeros((LANES,), jnp.float32)
        for j in range(EMB):
            t = plsc.load_gather(trows_v, [rows, perm[j]])
            c = plsc.load_gather(crows_v, [rows, perm[j]])
            acc = acc + t * c
        dots_v[pl.ds(rb, LANES)] = acc
        return ssq + acc * acc

    ssq = lax.fori_loop(0, per_w // LANES, blk,
                        jnp.zeros((LANES,), jnp.float32))
    part_v[...] = ssq
    pltpu.sync_copy(dots_v, dots_hbm.at[pl.ds(base, per_w)])
    pltpu.sync_copy(part_v, part_hbm.at[pl.ds(wid * LANES, LANES)])


@functools.partial(jax.jit, static_argnums=(3,))
def _sc_call(tgt, ctx, tbl, n):
    per_w = n // NW
    mesh = plsc.VectorSubcoreMesh(core_axis_name="c", subcore_axis_name="s")
    kern = functools.partial(
        pl.kernel,
        out_type=[
            jax.ShapeDtypeStruct((n,), jnp.float32),
            jax.ShapeDtypeStruct((NW * LANES,), jnp.float32),
        ],
        mesh=mesh,
        compiler_params=pltpu.CompilerParams(
            needs_layout_passes=False, use_tc_tiling_on_sc=False),
        scratch_types=[
            pltpu.VMEM((per_w,), jnp.int32),
            pltpu.VMEM((per_w,), jnp.int32),
            pltpu.VMEM((per_w, EMB), jnp.float32),
            pltpu.VMEM((per_w, EMB), jnp.float32),
            pltpu.VMEM((per_w,), jnp.float32),
            pltpu.VMEM((LANES,), jnp.float32),
            pltpu.SemaphoreType.DMA,
        ],
    )(_sc_body)
    return kern(tgt, ctx, tbl)


def _tc_body(dots_ref, part_ref, out_ref):
    ssq = jnp.sum(part_ref[...])
    scale = lax.rsqrt(jnp.maximum(ssq, 1e-12))
    out_ref[...] = dots_ref[...] * scale


def kernel(target, context, emb_table):
    b, l = target.shape
    n = b * l
    tgt = target.T.reshape(n).astype(jnp.int32)
    ctx = context.T.reshape(n).astype(jnp.int32)
    tblq = _tr_call(emb_table.T)
    dots, part = _sc_call(tgt, ctx, tblq.reshape(emb_table.shape[0], EMB), n)
    cos = pl.pallas_call(
        _tc_body,
        out_shape=jax.ShapeDtypeStruct((n // 128, 128), jnp.float32),
    )(dots.reshape(n // 128, 128), part.reshape(4, 128))
    return cos.reshape(l, b).T
